# baseline (speedup 1.0000x reference)
import functools

import jax
import jax.numpy as jnp
from jax import lax
from jax.experimental import pallas as pl
from jax.experimental.pallas import tpu as pltpu
from jax.experimental.pallas import tpu_sc as plsc

_NUM_WORKERS = 32
_NBUF = 4


def _sc_broadcast(pos_table, b, n, d):
    rows_per_worker = n // _NUM_WORKERS
    ch = 16
    nch = rows_per_worker // ch
    mesh = plsc.VectorSubcoreMesh(core_axis_name="c", subcore_axis_name="s")

    @functools.partial(
        pl.kernel,
        out_type=jax.ShapeDtypeStruct((b * n, d), pos_table.dtype),
        mesh=mesh,
        scratch_types=[
            pltpu.VMEM((_NBUF, ch, d), pos_table.dtype),
            pltpu.SemaphoreType.DMA((_NBUF,)),
            pltpu.SemaphoreType.DMA((_NBUF,)),
        ],
    )
    def k(tab_hbm, out_hbm, buf, sin, sout):
        wid = lax.axis_index("c") * 16 + lax.axis_index("s")
        base = wid * rows_per_worker

        def in_copy(c):
            return pltpu.make_async_copy(
                tab_hbm.at[pl.ds(base + c * ch, ch)],
                buf.at[c % _NBUF],
                sin.at[c % _NBUF],
            )

        def out_copy(c, bb):
            return pltpu.make_async_copy(
                buf.at[c % _NBUF],
                out_hbm.at[pl.ds(bb * n + base + c * ch, ch)],
                sout.at[c % _NBUF],
            )

        ins, outs = {}, {}
        for c in range(min(_NBUF, nch)):
            ins[c] = in_copy(c)
            ins[c].start()
        for c in range(nch):
            ins[c].wait()
            outs[c] = [out_copy(c, bb) for bb in range(b)]
            for h in outs[c]:
                h.start()
            if _NBUF <= c + 1 < nch:
                for h in outs[c + 1 - _NBUF]:
                    h.wait()
                ins[c + 1] = in_copy(c + 1)
                ins[c + 1].start()
        for c in range(max(0, nch - _NBUF), nch):
            for h in outs[c]:
                h.wait()

    return k(pos_table)


def _tc_copy_body(tab_ref, out_ref):
    t = tab_ref[...]
    for bb in range(out_ref.shape[0]):
        out_ref[bb] = t


def _tc_broadcast(pos_table, b, n, d):
    bn = 1024
    return pl.pallas_call(
        _tc_copy_body,
        grid=(n // bn,),
        in_specs=[pl.BlockSpec((bn, d), lambda i: (i, 0))],
        out_specs=pl.BlockSpec((b, bn, d), lambda i: (0, i, 0)),
        out_shape=jax.ShapeDtypeStruct((b, n, d), pos_table.dtype),
    )(pos_table)


def kernel(x, pos_table):
    b, n = x.shape[0], x.shape[1]
    d = pos_table.shape[1]
    out = _sc_broadcast(pos_table, b, n, d)
    return out.reshape(b, n, d)


# speedup vs baseline: 1.8741x; 1.8741x over previous
import functools

import jax
import jax.numpy as jnp
from jax import lax
from jax.experimental import pallas as pl
from jax.experimental.pallas import tpu as pltpu
from jax.experimental.pallas import tpu_sc as plsc

_NUM_WORKERS = 32
_NBUF = 4


def _sc_broadcast(pos_table, b, n, d):
    rows_per_worker = n // _NUM_WORKERS
    ch = 16
    nch = rows_per_worker // ch
    mesh = plsc.VectorSubcoreMesh(core_axis_name="c", subcore_axis_name="s")

    @functools.partial(
        pl.kernel,
        out_type=jax.ShapeDtypeStruct((b * n, d), pos_table.dtype),
        mesh=mesh,
        scratch_types=[
            pltpu.VMEM((_NBUF, ch, d), pos_table.dtype),
            pltpu.SemaphoreType.DMA((_NBUF,)),
            pltpu.SemaphoreType.DMA((_NBUF,)),
        ],
    )
    def k(tab_hbm, out_hbm, buf, sin, sout):
        wid = lax.axis_index("c") * 16 + lax.axis_index("s")
        base = wid * rows_per_worker

        def in_copy(c):
            return pltpu.make_async_copy(
                tab_hbm.at[pl.ds(base + c * ch, ch)],
                buf.at[c % _NBUF],
                sin.at[c % _NBUF],
            )

        def out_copy(c, bb):
            return pltpu.make_async_copy(
                buf.at[c % _NBUF],
                out_hbm.at[pl.ds(bb * n + base + c * ch, ch)],
                sout.at[c % _NBUF],
            )

        ins, outs = {}, {}
        for c in range(min(_NBUF, nch)):
            ins[c] = in_copy(c)
            ins[c].start()
        for c in range(nch):
            ins[c].wait()
            outs[c] = [out_copy(c, bb) for bb in range(b)]
            for h in outs[c]:
                h.start()
            if _NBUF <= c + 1 < nch:
                for h in outs[c + 1 - _NBUF]:
                    h.wait()
                ins[c + 1] = in_copy(c + 1)
                ins[c + 1].start()
        for c in range(max(0, nch - _NBUF), nch):
            for h in outs[c]:
                h.wait()

    return k(pos_table)


def _tc_manual(pos_table, b, n, d, ch=512, nbuf=4):
    nch = n // ch

    def body(tab_hbm, out_hbm, buf, sin, sout):
        def in_copy(c):
            return pltpu.make_async_copy(
                tab_hbm.at[pl.ds(c * ch, ch)],
                buf.at[c % nbuf],
                sin.at[c % nbuf],
            )

        def out_copy(c, bb):
            return pltpu.make_async_copy(
                buf.at[c % nbuf],
                out_hbm.at[pl.ds(bb * n + c * ch, ch)],
                sout.at[c % nbuf],
            )

        ins, outs = {}, {}
        for c in range(min(nbuf, nch)):
            ins[c] = in_copy(c)
            ins[c].start()
        for c in range(nch):
            ins[c].wait()
            outs[c] = [out_copy(c, bb) for bb in range(b)]
            for h in outs[c]:
                h.start()
            if nbuf <= c + 1 < nch:
                for h in outs[c + 1 - nbuf]:
                    h.wait()
                ins[c + 1] = in_copy(c + 1)
                ins[c + 1].start()
        for c in range(max(0, nch - nbuf), nch):
            for h in outs[c]:
                h.wait()

    return pl.pallas_call(
        body,
        in_specs=[pl.BlockSpec(memory_space=pltpu.MemorySpace.HBM)],
        out_specs=pl.BlockSpec(memory_space=pltpu.MemorySpace.HBM),
        out_shape=jax.ShapeDtypeStruct((b * n, d), pos_table.dtype),
        scratch_shapes=[
            pltpu.VMEM((nbuf, ch, d), pos_table.dtype),
            pltpu.SemaphoreType.DMA((nbuf,)),
            pltpu.SemaphoreType.DMA((nbuf,)),
        ],
    )(pos_table)


def _tc_copy_body(tab_ref, out_ref):
    t = tab_ref[...]
    for bb in range(out_ref.shape[0]):
        out_ref[bb] = t


def _tc_broadcast(pos_table, b, n, d):
    bn = 1024
    return pl.pallas_call(
        _tc_copy_body,
        grid=(n // bn,),
        in_specs=[pl.BlockSpec((bn, d), lambda i: (i, 0))],
        out_specs=pl.BlockSpec((b, bn, d), lambda i: (0, i, 0)),
        out_shape=jax.ShapeDtypeStruct((b, n, d), pos_table.dtype),
    )(pos_table)


def kernel(x, pos_table):
    b, n = x.shape[0], x.shape[1]
    d = pos_table.shape[1]
    out = _tc_manual(pos_table, b, n, d)
    return out.reshape(b, n, d)


# speedup vs baseline: 1.9375x; 1.0338x over previous
import functools

import jax
import jax.numpy as jnp
from jax import lax
from jax.experimental import pallas as pl
from jax.experimental.pallas import tpu as pltpu
from jax.experimental.pallas import tpu_sc as plsc

_NUM_WORKERS = 32
_NBUF = 4


def _sc_broadcast(pos_table, b, n, d):
    rows_per_worker = n // _NUM_WORKERS
    ch = 16
    nch = rows_per_worker // ch
    mesh = plsc.VectorSubcoreMesh(core_axis_name="c", subcore_axis_name="s")

    @functools.partial(
        pl.kernel,
        out_type=jax.ShapeDtypeStruct((b * n, d), pos_table.dtype),
        mesh=mesh,
        scratch_types=[
            pltpu.VMEM((_NBUF, ch, d), pos_table.dtype),
            pltpu.SemaphoreType.DMA((_NBUF,)),
            pltpu.SemaphoreType.DMA((_NBUF,)),
        ],
    )
    def k(tab_hbm, out_hbm, buf, sin, sout):
        wid = lax.axis_index("c") * 16 + lax.axis_index("s")
        base = wid * rows_per_worker

        def in_copy(c):
            return pltpu.make_async_copy(
                tab_hbm.at[pl.ds(base + c * ch, ch)],
                buf.at[c % _NBUF],
                sin.at[c % _NBUF],
            )

        def out_copy(c, bb):
            return pltpu.make_async_copy(
                buf.at[c % _NBUF],
                out_hbm.at[pl.ds(bb * n + base + c * ch, ch)],
                sout.at[c % _NBUF],
            )

        ins, outs = {}, {}
        for c in range(min(_NBUF, nch)):
            ins[c] = in_copy(c)
            ins[c].start()
        for c in range(nch):
            ins[c].wait()
            outs[c] = [out_copy(c, bb) for bb in range(b)]
            for h in outs[c]:
                h.start()
            if _NBUF <= c + 1 < nch:
                for h in outs[c + 1 - _NBUF]:
                    h.wait()
                ins[c + 1] = in_copy(c + 1)
                ins[c + 1].start()
        for c in range(max(0, nch - _NBUF), nch):
            for h in outs[c]:
                h.wait()

    return k(pos_table)


def _tc_manual(pos_table, b, n, d, ch=1024, nbuf=4):
    nch = n // ch

    def body(tab_hbm, out_hbm, buf, sin, sout):
        def in_copy(c):
            return pltpu.make_async_copy(
                tab_hbm.at[pl.ds(c * ch, ch)],
                buf.at[c % nbuf],
                sin.at[c % nbuf],
            )

        def out_copy(c, bb):
            return pltpu.make_async_copy(
                buf.at[c % nbuf],
                out_hbm.at[pl.ds(bb * n + c * ch, ch)],
                sout.at[c % nbuf],
            )

        ins, outs = {}, {}
        for c in range(min(nbuf, nch)):
            ins[c] = in_copy(c)
            ins[c].start()
        for c in range(nch):
            ins[c].wait()
            outs[c] = [out_copy(c, bb) for bb in range(b)]
            for h in outs[c]:
                h.start()
            if nbuf <= c + 1 < nch:
                for h in outs[c + 1 - nbuf]:
                    h.wait()
                ins[c + 1] = in_copy(c + 1)
                ins[c + 1].start()
        for c in range(max(0, nch - nbuf), nch):
            for h in outs[c]:
                h.wait()

    return pl.pallas_call(
        body,
        in_specs=[pl.BlockSpec(memory_space=pltpu.MemorySpace.HBM)],
        out_specs=pl.BlockSpec(memory_space=pltpu.MemorySpace.HBM),
        out_shape=jax.ShapeDtypeStruct((b * n, d), pos_table.dtype),
        scratch_shapes=[
            pltpu.VMEM((nbuf, ch, d), pos_table.dtype),
            pltpu.SemaphoreType.DMA((nbuf,)),
            pltpu.SemaphoreType.DMA((nbuf,)),
        ],
    )(pos_table)


def _tc_copy_body(tab_ref, out_ref):
    t = tab_ref[...]
    for bb in range(out_ref.shape[0]):
        out_ref[bb] = t


def _tc_broadcast(pos_table, b, n, d):
    bn = 1024
    return pl.pallas_call(
        _tc_copy_body,
        grid=(n // bn,),
        in_specs=[pl.BlockSpec((bn, d), lambda i: (i, 0))],
        out_specs=pl.BlockSpec((b, bn, d), lambda i: (0, i, 0)),
        out_shape=jax.ShapeDtypeStruct((b, n, d), pos_table.dtype),
    )(pos_table)


def kernel(x, pos_table):
    b, n = x.shape[0], x.shape[1]
    d = pos_table.shape[1]
    out = _tc_manual(pos_table, b, n, d)
    return out.reshape(b, n, d)
